# baseline (speedup 1.0000x reference)
import functools

import jax
import jax.numpy as jnp
from jax import lax
from jax.experimental import pallas as pl
from jax.experimental.pallas import tpu as pltpu
from jax.experimental.pallas import tpu_sc as plsc

E = 8
H = 1024
F = 2048
N = 2048
N2 = 2 * N
JITTER_EPS = 0.01

T = 256
G = N2 // T + E
SP = G * T
FK = 2
FC = F // FK
NT = N // T

INTERPRET = False


def _router_body(x_ref, gw_ref, logits_ref, oh1_ref, oh2_ref, m1_ref, m2_ref):
    x = x_ref[...]
    gw = gw_ref[...]
    logits = lax.dot_general(x, gw, (((1,), (1,)), ((), ())),
                             preferred_element_type=jnp.float32)
    logits_ref[...] = logits

    col = lax.broadcasted_iota(jnp.int32, logits.shape, 1)
    neg_inf = jnp.float32(-jnp.inf)

    t1 = jnp.max(logits, axis=-1, keepdims=True)
    idx1 = jnp.min(jnp.where(logits == t1, col, E), axis=-1, keepdims=True)
    is1 = (col == idx1)

    factor1 = jnp.maximum(jnp.abs(logits), t1)
    mask1 = ((t1 - logits) / factor1) > (2.0 * JITTER_EPS)
    masked1 = jnp.where(mask1, neg_inf, logits)
    z1 = masked1 - jnp.max(masked1, axis=-1, keepdims=True)
    e1 = jnp.exp(z1)
    mult1 = jnp.sum(jnp.where(is1, e1, 0.0), axis=-1, keepdims=True) \
        / jnp.sum(e1, axis=-1, keepdims=True)

    scattered = jnp.where(is1, neg_inf, logits)
    t2 = jnp.max(scattered, axis=-1, keepdims=True)
    idx2 = jnp.min(jnp.where(scattered == t2, col, E), axis=-1, keepdims=True)
    is2 = (col == idx2)

    factor2 = jnp.maximum(jnp.abs(logits), t2)
    mask2 = ((t2 - logits) / factor2) > (2.0 * JITTER_EPS)
    masked2 = jnp.where(mask2, neg_inf, scattered)
    z2 = masked2 - jnp.max(masked2, axis=-1, keepdims=True)
    e2 = jnp.exp(z2)
    mult2 = jnp.sum(jnp.where(is2, e2, 0.0), axis=-1, keepdims=True) \
        / jnp.sum(e2, axis=-1, keepdims=True)

    oh1_ref[...] = is1.astype(jnp.float32)
    oh2_ref[...] = is2.astype(jnp.float32)
    m1_ref[...] = mult1
    m2_ref[...] = mult2


def _run_router(x, gate_w):
    return pl.pallas_call(
        _router_body,
        grid=(NT,),
        in_specs=[
            pl.BlockSpec((T, H), lambda t: (t, 0)),
            pl.BlockSpec((E, H), lambda t: (0, 0)),
        ],
        out_specs=[
            pl.BlockSpec((T, E), lambda t: (t, 0)),
            pl.BlockSpec((T, E), lambda t: (t, 0)),
            pl.BlockSpec((T, E), lambda t: (t, 0)),
            pl.BlockSpec((T, 1), lambda t: (t, 0)),
            pl.BlockSpec((T, 1), lambda t: (t, 0)),
        ],
        out_shape=[
            jax.ShapeDtypeStruct((N, E), jnp.float32),
            jax.ShapeDtypeStruct((N, E), jnp.float32),
            jax.ShapeDtypeStruct((N, E), jnp.float32),
            jax.ShapeDtypeStruct((N, 1), jnp.float32),
            jax.ShapeDtypeStruct((N, 1), jnp.float32),
        ],
        interpret=INTERPRET,
    )(x, gate_w)


_CHUNK = 512
_NCH = N2 // _CHUNK


def _dispatch_body(oh1_ref, oh2_ref, pos_ref, te_ref, gmap_ref, slot_ref):
    a = jnp.concatenate([oh1_ref[...], oh2_ref[...]], axis=0)

    r = lax.broadcasted_iota(jnp.int32, (_CHUNK, _CHUNK), 0)
    c = lax.broadcasted_iota(jnp.int32, (_CHUNK, _CHUNK), 1)
    ltri = (r >= c).astype(jnp.float32)
    carry = jnp.zeros((1, E), jnp.float32)
    parts = []
    for ci in range(_NCH):
        blk = a[ci * _CHUNK:(ci + 1) * _CHUNK]
        local = lax.dot_general(ltri, blk, (((1,), (0,)), ((), ())),
                                preferred_element_type=jnp.float32)
        parts.append(local + carry)
        carry = carry + jnp.sum(blk, axis=0, keepdims=True)
    c_incl = jnp.concatenate(parts, axis=0)

    counts = carry
    pc = jnp.floor((counts + (T - 1)) * (1.0 / T)) * T
    ntiles = jnp.sum(pc) * (1.0 / T)

    er = lax.broadcasted_iota(jnp.int32, (E, E), 0)
    ec = lax.broadcasted_iota(jnp.int32, (E, E), 1)
    ident = (er == ec).astype(jnp.float32)
    strict_u = (er < ec).astype(jnp.float32)
    base_row = lax.dot_general(pc, strict_u, (((1,), (0,)), ((), ())),
                               preferred_element_type=jnp.float32)
    base_col = lax.dot_general(ident, base_row, (((1,), (1,)), ((), ())),
                               preferred_element_type=jnp.float32)

    rank = jnp.sum(a * c_incl, axis=1, keepdims=True) - 1.0
    base_of_j = jnp.sum(a * base_row, axis=1, keepdims=True)
    pos_ref[...] = (rank + base_of_j).astype(jnp.int32)

    baseT_col = base_col * (1.0 / T)
    giota = lax.broadcasted_iota(jnp.int32, (1, G), 1).astype(jnp.float32)
    gvals = jnp.minimum(giota, ntiles - 1.0)
    cmp = (baseT_col <= gvals).astype(jnp.float32)
    te = jnp.sum(cmp, axis=0, keepdims=True) - 1.0
    te_ref[...] = te.astype(jnp.int32)
    gmap_ref[...] = gvals.astype(jnp.int32)

    ne_row = (counts >= 0.5).astype(jnp.float32)
    ne_col = lax.dot_general(ident, ne_row, (((1,), (1,)), ((), ())),
                             preferred_element_type=jnp.float32)
    changes = jnp.sum(ne_col * cmp, axis=0, keepdims=True)
    slot = changes - 1.0
    slot_ref[...] = (slot - 2.0 * jnp.floor(slot * 0.5)).astype(jnp.int32)


def _run_dispatch(oh1, oh2):
    return pl.pallas_call(
        _dispatch_body,
        grid=(1,),
        in_specs=[
            pl.BlockSpec((N, E), lambda i: (0, 0)),
            pl.BlockSpec((N, E), lambda i: (0, 0)),
        ],
        out_specs=[
            pl.BlockSpec((N2, 1), lambda i: (0, 0)),
            pl.BlockSpec((1, G), lambda i: (0, 0)),
            pl.BlockSpec((1, G), lambda i: (0, 0)),
            pl.BlockSpec((1, G), lambda i: (0, 0)),
        ],
        out_shape=[
            jax.ShapeDtypeStruct((N2, 1), jnp.int32),
            jax.ShapeDtypeStruct((1, G), jnp.int32),
            jax.ShapeDtypeStruct((1, G), jnp.int32),
            jax.ShapeDtypeStruct((1, G), jnp.int32),
        ],
        interpret=INTERPRET,
    )(oh1, oh2)


_ROWCHUNK = 32


def _make_sc_scatter():
    mesh = plsc.VectorSubcoreMesh(core_axis_name="c", subcore_axis_name="s")
    info = plsc.get_sparse_core_info()
    nc, ns = info.num_cores, info.num_subcores
    nw = nc * ns
    per_w = N // nw
    nch = per_w // _ROWCHUNK

    @functools.partial(
        pl.kernel,
        mesh=mesh,
        out_type=jax.ShapeDtypeStruct((SP, H), jnp.float32),
        scratch_types=[
            pltpu.VMEM((_ROWCHUNK,), jnp.int32),
            pltpu.VMEM((_ROWCHUNK,), jnp.int32),
            pltpu.VMEM((_ROWCHUNK, H), jnp.float32),
            pltpu.SemaphoreType.DMA,
        ],
    )
    def scatter_kernel(x_hbm, pos_hbm, xs_hbm, idx1_v, idx2_v, rows_v, sem):
        wid = lax.axis_index("s") * nc + lax.axis_index("c")
        base = wid * per_w
        for ch in range(nch):
            o = base + ch * _ROWCHUNK
            pltpu.sync_copy(x_hbm.at[pl.ds(o, _ROWCHUNK)], rows_v)
            pltpu.sync_copy(pos_hbm.at[pl.ds(o, _ROWCHUNK)], idx1_v)
            pltpu.sync_copy(pos_hbm.at[pl.ds(N + o, _ROWCHUNK)], idx2_v)
            pltpu.async_copy(rows_v, xs_hbm.at[idx1_v], sem).wait()
            pltpu.async_copy(rows_v, xs_hbm.at[idx2_v], sem).wait()

    return scatter_kernel


_WCH = 4

def _ffn_body(te_ref, gmap_ref, slot_ref, xs_ref, w1_hbm, w3_hbm, w2_hbm,
              ys_ref, w1s, w3s, w2s, w1b, w3b, w2b, s1, s3, s2):
    g = pl.program_id(0)
    cur = te_ref[g]
    slot = slot_ref[g]
    prv = jnp.where(g == 0, -1, te_ref[jnp.maximum(g - 1, 0)])
    nxt = te_ref[jnp.minimum(g + 1, G - 1)]

    def _w_copies(e):
        cps = []
        fc = F // _WCH
        hc = H // _WCH
        for k in range(_WCH):
            cps.append(pltpu.make_async_copy(
                w1_hbm.at[e, pl.ds(k * fc, fc)], w1s.at[pl.ds(k * fc, fc)],
                s1.at[k]))
            cps.append(pltpu.make_async_copy(
                w3_hbm.at[e, pl.ds(k * fc, fc)], w3s.at[pl.ds(k * fc, fc)],
                s3.at[k]))
            cps.append(pltpu.make_async_copy(
                w2_hbm.at[e, pl.ds(k * hc, hc)], w2s.at[pl.ds(k * hc, hc)],
                s2.at[k]))
        return cps

    @pl.when(g == 0)
    def _():
        for cp in _w_copies(cur):
            cp.start()

    @pl.when(cur != prv)
    def _():
        for cp in _w_copies(cur):
            cp.wait()
        w1b[slot] = w1s[...].astype(jnp.bfloat16)
        w3b[slot] = w3s[...].astype(jnp.bfloat16)
        w2b[slot] = w2s[...].astype(jnp.bfloat16)

    @pl.when(jnp.logical_and(g + 1 < G, nxt != cur))
    def _():
        for cp in _w_copies(nxt):
            cp.start()

    @pl.when(gmap_ref[g] == g)
    def _():
        xs = xs_ref[...].astype(jnp.bfloat16)
        w1c = w1b[slot]
        w3c = w3b[slot]
        w2c = w2b[slot]
        h1 = lax.dot_general(xs, w1c, (((1,), (1,)), ((), ())),
                             preferred_element_type=jnp.float32)
        h3 = lax.dot_general(xs, w3c, (((1,), (1,)), ((), ())),
                             preferred_element_type=jnp.float32)
        act = ((h1 / (1.0 + jnp.exp(-h1))) * h3).astype(jnp.bfloat16)
        ys_ref[...] = lax.dot_general(act, w2c, (((1,), (1,)), ((), ())),
                                      preferred_element_type=jnp.float32)


def _run_ffn(xs, te, gmap, slot, w1, w3, w2):
    grid_spec = pltpu.PrefetchScalarGridSpec(
        num_scalar_prefetch=3,
        grid=(G,),
        in_specs=[
            pl.BlockSpec((T, H), lambda g, te, gm, sl: (gm[g], 0)),
            pl.BlockSpec(memory_space=pltpu.MemorySpace.HBM),
            pl.BlockSpec(memory_space=pltpu.MemorySpace.HBM),
            pl.BlockSpec(memory_space=pltpu.MemorySpace.HBM),
        ],
        out_specs=pl.BlockSpec((T, H), lambda g, te, gm, sl: (gm[g], 0)),
        scratch_shapes=[
            pltpu.VMEM((F, H), jnp.float32),
            pltpu.VMEM((F, H), jnp.float32),
            pltpu.VMEM((H, F), jnp.float32),
            pltpu.VMEM((2, F, H), jnp.bfloat16),
            pltpu.VMEM((2, F, H), jnp.bfloat16),
            pltpu.VMEM((2, H, F), jnp.bfloat16),
            pltpu.SemaphoreType.DMA((_WCH,)),
            pltpu.SemaphoreType.DMA((_WCH,)),
            pltpu.SemaphoreType.DMA((_WCH,)),
        ],
    )
    return pl.pallas_call(
        _ffn_body,
        grid_spec=grid_spec,
        out_shape=jax.ShapeDtypeStruct((SP, H), jnp.float32),
        interpret=INTERPRET,
    )(te, gmap, slot, xs, w1, w3, w2)


def _make_sc_gather():
    mesh = plsc.VectorSubcoreMesh(core_axis_name="c", subcore_axis_name="s")
    info = plsc.get_sparse_core_info()
    nc, ns = info.num_cores, info.num_subcores
    nw = nc * ns
    per_w = N // nw
    nch = per_w // _ROWCHUNK

    @functools.partial(
        pl.kernel,
        mesh=mesh,
        out_type=[
            jax.ShapeDtypeStruct((N, H), jnp.float32),
            jax.ShapeDtypeStruct((N, H), jnp.float32),
        ],
        scratch_types=[
            pltpu.VMEM((_ROWCHUNK,), jnp.int32),
            pltpu.VMEM((_ROWCHUNK,), jnp.int32),
            pltpu.VMEM((_ROWCHUNK, H), jnp.float32),
            pltpu.VMEM((_ROWCHUNK, H), jnp.float32),
            pltpu.SemaphoreType.DMA,
        ],
    )
    def gather_kernel(ys_hbm, pos_hbm, y1_hbm, y2_hbm,
                      idx1_v, idx2_v, rows1_v, rows2_v, sem):
        wid = lax.axis_index("s") * nc + lax.axis_index("c")
        base = wid * per_w
        for ch in range(nch):
            o = base + ch * _ROWCHUNK
            pltpu.sync_copy(pos_hbm.at[pl.ds(o, _ROWCHUNK)], idx1_v)
            pltpu.sync_copy(pos_hbm.at[pl.ds(N + o, _ROWCHUNK)], idx2_v)
            pltpu.async_copy(ys_hbm.at[idx1_v], rows1_v, sem).wait()
            pltpu.async_copy(ys_hbm.at[idx2_v], rows2_v, sem).wait()
            pltpu.sync_copy(rows1_v, y1_hbm.at[pl.ds(o, _ROWCHUNK)])
            pltpu.sync_copy(rows2_v, y2_hbm.at[pl.ds(o, _ROWCHUNK)])

    return gather_kernel


def _combine_body(y1_ref, y2_ref, m1_ref, m2_ref, out_ref):
    out_ref[...] = y1_ref[...] * m1_ref[...] + y2_ref[...] * m2_ref[...]


def _run_combine(y1, y2, m1, m2):
    return pl.pallas_call(
        _combine_body,
        grid=(NT,),
        in_specs=[
            pl.BlockSpec((T, H), lambda t: (t, 0)),
            pl.BlockSpec((T, H), lambda t: (t, 0)),
            pl.BlockSpec((T, 1), lambda t: (t, 0)),
            pl.BlockSpec((T, 1), lambda t: (t, 0)),
        ],
        out_specs=pl.BlockSpec((T, H), lambda t: (t, 0)),
        out_shape=jax.ShapeDtypeStruct((N, H), jnp.float32),
        interpret=INTERPRET,
    )(y1, y2, m1, m2)


def kernel(hidden_states, gate_w, w1, w2, w3):
    B, S, Hd = hidden_states.shape
    x = hidden_states.reshape(-1, Hd)
    logits, oh1, oh2, m1, m2 = _run_router(x, gate_w)
    pos2, te2, gmap2, slot2 = _run_dispatch(oh1, oh2)
    pos = pos2.reshape(N2)
    te = te2.reshape(G)
    gmap = gmap2.reshape(G)
    slot = slot2.reshape(G)
    xs = _make_sc_scatter()(x, pos)
    ys = _run_ffn(xs, te, gmap, slot, w1, w3, w2)
    y1, y2 = _make_sc_gather()(ys, pos)
    out = _run_combine(y1, y2, m1, m2)
    return out.reshape(B, S, Hd), logits


# speedup vs baseline: 1.0946x; 1.0946x over previous
import functools

import jax
import jax.numpy as jnp
from jax import lax
from jax.experimental import pallas as pl
from jax.experimental.pallas import tpu as pltpu
from jax.experimental.pallas import tpu_sc as plsc

E = 8
H = 1024
F = 2048
N = 2048
N2 = 2 * N
JITTER_EPS = 0.01

T = 256
G = N2 // T + E
SP = G * T
FK = 2
FC = F // FK
NT = N // T

INTERPRET = False


def _router_body(x_ref, gw_ref, logits_ref, oh1_ref, oh2_ref, m1_ref, m2_ref):
    x = x_ref[...]
    gw = gw_ref[...]
    logits = lax.dot_general(x, gw, (((1,), (1,)), ((), ())),
                             preferred_element_type=jnp.float32)
    logits_ref[...] = logits

    col = lax.broadcasted_iota(jnp.int32, logits.shape, 1)
    neg_inf = jnp.float32(-jnp.inf)

    t1 = jnp.max(logits, axis=-1, keepdims=True)
    idx1 = jnp.min(jnp.where(logits == t1, col, E), axis=-1, keepdims=True)
    is1 = (col == idx1)

    factor1 = jnp.maximum(jnp.abs(logits), t1)
    mask1 = ((t1 - logits) / factor1) > (2.0 * JITTER_EPS)
    masked1 = jnp.where(mask1, neg_inf, logits)
    z1 = masked1 - jnp.max(masked1, axis=-1, keepdims=True)
    e1 = jnp.exp(z1)
    mult1 = jnp.sum(jnp.where(is1, e1, 0.0), axis=-1, keepdims=True) \
        / jnp.sum(e1, axis=-1, keepdims=True)

    scattered = jnp.where(is1, neg_inf, logits)
    t2 = jnp.max(scattered, axis=-1, keepdims=True)
    idx2 = jnp.min(jnp.where(scattered == t2, col, E), axis=-1, keepdims=True)
    is2 = (col == idx2)

    factor2 = jnp.maximum(jnp.abs(logits), t2)
    mask2 = ((t2 - logits) / factor2) > (2.0 * JITTER_EPS)
    masked2 = jnp.where(mask2, neg_inf, scattered)
    z2 = masked2 - jnp.max(masked2, axis=-1, keepdims=True)
    e2 = jnp.exp(z2)
    mult2 = jnp.sum(jnp.where(is2, e2, 0.0), axis=-1, keepdims=True) \
        / jnp.sum(e2, axis=-1, keepdims=True)

    oh1_ref[...] = is1.astype(jnp.float32)
    oh2_ref[...] = is2.astype(jnp.float32)
    m1_ref[...] = mult1
    m2_ref[...] = mult2


def _run_router(x, gate_w):
    return pl.pallas_call(
        _router_body,
        grid=(NT,),
        in_specs=[
            pl.BlockSpec((T, H), lambda t: (t, 0)),
            pl.BlockSpec((E, H), lambda t: (0, 0)),
        ],
        out_specs=[
            pl.BlockSpec((T, E), lambda t: (t, 0)),
            pl.BlockSpec((T, E), lambda t: (t, 0)),
            pl.BlockSpec((T, E), lambda t: (t, 0)),
            pl.BlockSpec((T, 1), lambda t: (t, 0)),
            pl.BlockSpec((T, 1), lambda t: (t, 0)),
        ],
        out_shape=[
            jax.ShapeDtypeStruct((N, E), jnp.float32),
            jax.ShapeDtypeStruct((N, E), jnp.float32),
            jax.ShapeDtypeStruct((N, E), jnp.float32),
            jax.ShapeDtypeStruct((N, 1), jnp.float32),
            jax.ShapeDtypeStruct((N, 1), jnp.float32),
        ],
        interpret=INTERPRET,
    )(x, gate_w)


_CHUNK = 512
_NCH = N2 // _CHUNK


def _dispatch_body(oh1_ref, oh2_ref, pos_ref, te_ref, gmap_ref, slot_ref,
                   nde_ref):
    a = jnp.concatenate([oh1_ref[...], oh2_ref[...]], axis=0)

    r = lax.broadcasted_iota(jnp.int32, (_CHUNK, _CHUNK), 0)
    c = lax.broadcasted_iota(jnp.int32, (_CHUNK, _CHUNK), 1)
    ltri = (r >= c).astype(jnp.float32)
    carry = jnp.zeros((1, E), jnp.float32)
    parts = []
    for ci in range(_NCH):
        blk = a[ci * _CHUNK:(ci + 1) * _CHUNK]
        local = lax.dot_general(ltri, blk, (((1,), (0,)), ((), ())),
                                preferred_element_type=jnp.float32)
        parts.append(local + carry)
        carry = carry + jnp.sum(blk, axis=0, keepdims=True)
    c_incl = jnp.concatenate(parts, axis=0)

    counts = carry
    pc = jnp.floor((counts + (T - 1)) * (1.0 / T)) * T
    ntiles = jnp.sum(pc) * (1.0 / T)

    er = lax.broadcasted_iota(jnp.int32, (E, E), 0)
    ec = lax.broadcasted_iota(jnp.int32, (E, E), 1)
    ident = (er == ec).astype(jnp.float32)
    strict_u = (er < ec).astype(jnp.float32)
    base_row = lax.dot_general(pc, strict_u, (((1,), (0,)), ((), ())),
                               preferred_element_type=jnp.float32)
    base_col = lax.dot_general(ident, base_row, (((1,), (1,)), ((), ())),
                               preferred_element_type=jnp.float32)

    rank = jnp.sum(a * c_incl, axis=1, keepdims=True) - 1.0
    base_of_j = jnp.sum(a * base_row, axis=1, keepdims=True)
    pos_ref[...] = (rank + base_of_j).astype(jnp.int32)

    baseT_col = base_col * (1.0 / T)
    giota = lax.broadcasted_iota(jnp.int32, (1, G), 1).astype(jnp.float32)
    gvals = jnp.minimum(giota, ntiles - 1.0)
    cmp = (baseT_col <= gvals).astype(jnp.float32)
    te = jnp.sum(cmp, axis=0, keepdims=True) - 1.0
    te_ref[...] = te.astype(jnp.int32)
    gmap_ref[...] = gvals.astype(jnp.int32)

    ne_row = (counts >= 0.5).astype(jnp.float32)
    ne_col = lax.dot_general(ident, ne_row, (((1,), (1,)), ((), ())),
                             preferred_element_type=jnp.float32)
    changes = jnp.sum(ne_col * cmp, axis=0, keepdims=True)
    slot = changes - 1.0
    slot_ref[...] = (slot - 2.0 * jnp.floor(slot * 0.5)).astype(jnp.int32)

    pc_col = lax.dot_general(ident, pc, (((1,), (1,)), ((), ())),
                             preferred_element_type=jnp.float32)
    pcT_col = pc_col * (1.0 / T)
    endT_col = baseT_col + pcT_col
    oh = cmp * (gvals < endT_col).astype(jnp.float32)
    seg_end = jnp.sum(oh * endT_col, axis=0, keepdims=True)
    nxt_tile = jnp.minimum(seg_end, ntiles - 1.0)
    nde = jnp.sum((baseT_col <= nxt_tile).astype(jnp.float32),
                  axis=0, keepdims=True) - 1.0
    nde_ref[...] = nde.astype(jnp.int32)


def _run_dispatch(oh1, oh2):
    return pl.pallas_call(
        _dispatch_body,
        grid=(1,),
        in_specs=[
            pl.BlockSpec((N, E), lambda i: (0, 0)),
            pl.BlockSpec((N, E), lambda i: (0, 0)),
        ],
        out_specs=[
            pl.BlockSpec((N2, 1), lambda i: (0, 0)),
            pl.BlockSpec((1, G), lambda i: (0, 0)),
            pl.BlockSpec((1, G), lambda i: (0, 0)),
            pl.BlockSpec((1, G), lambda i: (0, 0)),
            pl.BlockSpec((1, G), lambda i: (0, 0)),
        ],
        out_shape=[
            jax.ShapeDtypeStruct((N2, 1), jnp.int32),
            jax.ShapeDtypeStruct((1, G), jnp.int32),
            jax.ShapeDtypeStruct((1, G), jnp.int32),
            jax.ShapeDtypeStruct((1, G), jnp.int32),
            jax.ShapeDtypeStruct((1, G), jnp.int32),
        ],
        interpret=INTERPRET,
    )(oh1, oh2)


_ROWCHUNK = 32


def _make_sc_scatter():
    mesh = plsc.VectorSubcoreMesh(core_axis_name="c", subcore_axis_name="s")
    info = plsc.get_sparse_core_info()
    nc, ns = info.num_cores, info.num_subcores
    nw = nc * ns
    per_w = N // nw
    nch = per_w // _ROWCHUNK

    @functools.partial(
        pl.kernel,
        mesh=mesh,
        out_type=jax.ShapeDtypeStruct((SP, H), jnp.float32),
        scratch_types=[
            pltpu.VMEM((_ROWCHUNK,), jnp.int32),
            pltpu.VMEM((_ROWCHUNK,), jnp.int32),
            pltpu.VMEM((_ROWCHUNK, H), jnp.float32),
            pltpu.SemaphoreType.DMA,
        ],
    )
    def scatter_kernel(x_hbm, pos_hbm, xs_hbm, idx1_v, idx2_v, rows_v, sem):
        wid = lax.axis_index("s") * nc + lax.axis_index("c")
        base = wid * per_w
        for ch in range(nch):
            o = base + ch * _ROWCHUNK
            pltpu.sync_copy(x_hbm.at[pl.ds(o, _ROWCHUNK)], rows_v)
            pltpu.sync_copy(pos_hbm.at[pl.ds(o, _ROWCHUNK)], idx1_v)
            pltpu.sync_copy(pos_hbm.at[pl.ds(N + o, _ROWCHUNK)], idx2_v)
            pltpu.async_copy(rows_v, xs_hbm.at[idx1_v], sem).wait()
            pltpu.async_copy(rows_v, xs_hbm.at[idx2_v], sem).wait()

    return scatter_kernel


_WCH = 4

def _ffn_body(te_ref, gmap_ref, slot_ref, nde_ref, xs_ref,
              w1_hbm, w3_hbm, w2_hbm,
              ys_ref, w1s, w3s, w2s, w1b, w3b, w2b, s1, s3, s2):
    g = pl.program_id(0)
    cur = te_ref[g]
    slot = slot_ref[g]
    prv = jnp.where(g == 0, -1, te_ref[jnp.maximum(g - 1, 0)])
    nde = nde_ref[g]

    def _w13_copies(e):
        cps = []
        fc = F // _WCH
        for k in range(_WCH):
            cps.append(pltpu.make_async_copy(
                w1_hbm.at[e, pl.ds(k * fc, fc)], w1s.at[pl.ds(k * fc, fc)],
                s1.at[k]))
            cps.append(pltpu.make_async_copy(
                w3_hbm.at[e, pl.ds(k * fc, fc)], w3s.at[pl.ds(k * fc, fc)],
                s3.at[k]))
        return cps

    def _w2_copy(e, k):
        fc = F // _WCH
        return pltpu.make_async_copy(
            w2_hbm.at[e, :, pl.ds(k * fc, fc)], w2s.at[:, pl.ds(k * fc, fc)],
            s2.at[k])

    @pl.when(g == 0)
    def _():
        for cp in _w13_copies(cur):
            cp.start()
        for k in range(_WCH):
            _w2_copy(cur, k).start()

    @pl.when(cur != prv)
    def _():
        for cp in _w13_copies(cur):
            cp.wait()
        fc = F // _WCH
        for k in range(_WCH):
            w1b[pl.ds(k * fc, fc), :] = \
                w1s[pl.ds(k * fc, fc), :].astype(jnp.bfloat16)
            w3b[pl.ds(k * fc, fc), :] = \
                w3s[pl.ds(k * fc, fc), :].astype(jnp.bfloat16)

        @pl.when(nde != cur)
        def _():
            for cp in _w13_copies(nde):
                cp.start()

    @pl.when(gmap_ref[g] == g)
    def _():
        xs = xs_ref[...].astype(jnp.bfloat16)
        acc = None
        kw = _WCH // FK
        for fk in range(FK):
            @pl.when(cur != prv)
            def _(fk=fk):
                for k in range(kw * fk, kw * (fk + 1)):
                    _w2_copy(cur, k).wait()
            w1c = w1b[pl.ds(fk * FC, FC), :]
            w3c = w3b[pl.ds(fk * FC, FC), :]
            w2c = w2s[:, pl.ds(fk * FC, FC)]
            h1 = lax.dot_general(xs, w1c, (((1,), (1,)), ((), ())),
                                 preferred_element_type=jnp.float32)
            h3 = lax.dot_general(xs, w3c, (((1,), (1,)), ((), ())),
                                 preferred_element_type=jnp.float32)
            act = (h1 / (1.0 + jnp.exp(-h1))) * h3
            part = lax.dot_general(act, w2c, (((1,), (1,)), ((), ())),
                                   preferred_element_type=jnp.float32)
            acc = part if acc is None else acc + part
        ys_ref[...] = acc

    @pl.when(jnp.logical_and(g + 1 < G,
                             te_ref[jnp.minimum(g + 1, G - 1)] != cur))
    def _():
        for k in range(_WCH):
            _w2_copy(nde, k).start()


def _run_ffn(xs, te, gmap, slot, nde, w1, w3, w2):
    grid_spec = pltpu.PrefetchScalarGridSpec(
        num_scalar_prefetch=4,
        grid=(G,),
        in_specs=[
            pl.BlockSpec((T, H), lambda g, te, gm, sl, nd: (gm[g], 0)),
            pl.BlockSpec(memory_space=pltpu.MemorySpace.HBM),
            pl.BlockSpec(memory_space=pltpu.MemorySpace.HBM),
            pl.BlockSpec(memory_space=pltpu.MemorySpace.HBM),
        ],
        out_specs=pl.BlockSpec((T, H), lambda g, te, gm, sl, nd: (gm[g], 0)),
        scratch_shapes=[
            pltpu.VMEM((F, H), jnp.float32),
            pltpu.VMEM((F, H), jnp.float32),
            pltpu.VMEM((H, F), jnp.float32),
            pltpu.VMEM((F, H), jnp.bfloat16),
            pltpu.VMEM((F, H), jnp.bfloat16),
            pltpu.VMEM((H, F), jnp.bfloat16),
            pltpu.SemaphoreType.DMA((_WCH,)),
            pltpu.SemaphoreType.DMA((_WCH,)),
            pltpu.SemaphoreType.DMA((_WCH,)),
        ],
    )
    return pl.pallas_call(
        _ffn_body,
        grid_spec=grid_spec,
        out_shape=jax.ShapeDtypeStruct((SP, H), jnp.float32),
        interpret=INTERPRET,
    )(te, gmap, slot, nde, xs, w1, w3, w2)


def _make_sc_gather():
    mesh = plsc.VectorSubcoreMesh(core_axis_name="c", subcore_axis_name="s")
    info = plsc.get_sparse_core_info()
    nc, ns = info.num_cores, info.num_subcores
    nw = nc * ns
    per_w = N // nw
    nch = per_w // _ROWCHUNK

    @functools.partial(
        pl.kernel,
        mesh=mesh,
        out_type=[
            jax.ShapeDtypeStruct((N, H), jnp.float32),
            jax.ShapeDtypeStruct((N, H), jnp.float32),
        ],
        scratch_types=[
            pltpu.VMEM((_ROWCHUNK,), jnp.int32),
            pltpu.VMEM((_ROWCHUNK,), jnp.int32),
            pltpu.VMEM((_ROWCHUNK, H), jnp.float32),
            pltpu.VMEM((_ROWCHUNK, H), jnp.float32),
            pltpu.SemaphoreType.DMA,
        ],
    )
    def gather_kernel(ys_hbm, pos_hbm, y1_hbm, y2_hbm,
                      idx1_v, idx2_v, rows1_v, rows2_v, sem):
        wid = lax.axis_index("s") * nc + lax.axis_index("c")
        base = wid * per_w
        for ch in range(nch):
            o = base + ch * _ROWCHUNK
            pltpu.sync_copy(pos_hbm.at[pl.ds(o, _ROWCHUNK)], idx1_v)
            pltpu.sync_copy(pos_hbm.at[pl.ds(N + o, _ROWCHUNK)], idx2_v)
            pltpu.async_copy(ys_hbm.at[idx1_v], rows1_v, sem).wait()
            pltpu.async_copy(ys_hbm.at[idx2_v], rows2_v, sem).wait()
            pltpu.sync_copy(rows1_v, y1_hbm.at[pl.ds(o, _ROWCHUNK)])
            pltpu.sync_copy(rows2_v, y2_hbm.at[pl.ds(o, _ROWCHUNK)])

    return gather_kernel


def _combine_body(y1_ref, y2_ref, m1_ref, m2_ref, out_ref):
    y1 = y1_ref[...].astype(jnp.float32)
    y2 = y2_ref[...].astype(jnp.float32)
    out_ref[...] = y1 * m1_ref[...] + y2 * m2_ref[...]


def _run_combine(y1, y2, m1, m2):
    return pl.pallas_call(
        _combine_body,
        grid=(NT,),
        in_specs=[
            pl.BlockSpec((T, H), lambda t: (t, 0)),
            pl.BlockSpec((T, H), lambda t: (t, 0)),
            pl.BlockSpec((T, 1), lambda t: (t, 0)),
            pl.BlockSpec((T, 1), lambda t: (t, 0)),
        ],
        out_specs=pl.BlockSpec((T, H), lambda t: (t, 0)),
        out_shape=jax.ShapeDtypeStruct((N, H), jnp.float32),
        interpret=INTERPRET,
    )(y1, y2, m1, m2)


def kernel(hidden_states, gate_w, w1, w2, w3):
    B, S, Hd = hidden_states.shape
    x = hidden_states.reshape(-1, Hd)
    logits, oh1, oh2, m1, m2 = _run_router(x, gate_w)
    pos2, te2, gmap2, slot2, nde2 = _run_dispatch(oh1, oh2)
    pos = pos2.reshape(N2)
    te = te2.reshape(G)
    gmap = gmap2.reshape(G)
    slot = slot2.reshape(G)
    nde = nde2.reshape(G)
    xs = _make_sc_scatter()(x, pos)
    ys = _run_ffn(xs, te, gmap, slot, nde, w1, w3, w2)
    y1, y2 = _make_sc_gather()(ys, pos)
    out = _run_combine(y1, y2, m1, m2)
    return out.reshape(B, S, Hd), logits


# speedup vs baseline: 1.0980x; 1.0031x over previous
import functools

import jax
import jax.numpy as jnp
from jax import lax
from jax.experimental import pallas as pl
from jax.experimental.pallas import tpu as pltpu
from jax.experimental.pallas import tpu_sc as plsc

E = 8
H = 1024
F = 2048
N = 2048
N2 = 2 * N
JITTER_EPS = 0.01

T = 256
G = N2 // T + E
SP = G * T
FK = 2
FC = F // FK
NT = N // T


def _router_body(x_ref, gw_ref, logits_ref, oh1_ref, oh2_ref, m1_ref, m2_ref):
    x = x_ref[...]
    gw = gw_ref[...]
    logits = lax.dot_general(x, gw, (((1,), (1,)), ((), ())),
                             preferred_element_type=jnp.float32)
    logits_ref[...] = logits

    col = lax.broadcasted_iota(jnp.int32, logits.shape, 1)
    neg_inf = jnp.float32(-jnp.inf)

    t1 = jnp.max(logits, axis=-1, keepdims=True)
    idx1 = jnp.min(jnp.where(logits == t1, col, E), axis=-1, keepdims=True)
    is1 = (col == idx1)

    factor1 = jnp.maximum(jnp.abs(logits), t1)
    mask1 = ((t1 - logits) / factor1) > (2.0 * JITTER_EPS)
    masked1 = jnp.where(mask1, neg_inf, logits)
    z1 = masked1 - jnp.max(masked1, axis=-1, keepdims=True)
    e1 = jnp.exp(z1)
    mult1 = jnp.sum(jnp.where(is1, e1, 0.0), axis=-1, keepdims=True) \
        / jnp.sum(e1, axis=-1, keepdims=True)

    scattered = jnp.where(is1, neg_inf, logits)
    t2 = jnp.max(scattered, axis=-1, keepdims=True)
    idx2 = jnp.min(jnp.where(scattered == t2, col, E), axis=-1, keepdims=True)
    is2 = (col == idx2)

    factor2 = jnp.maximum(jnp.abs(logits), t2)
    mask2 = ((t2 - logits) / factor2) > (2.0 * JITTER_EPS)
    masked2 = jnp.where(mask2, neg_inf, scattered)
    z2 = masked2 - jnp.max(masked2, axis=-1, keepdims=True)
    e2 = jnp.exp(z2)
    mult2 = jnp.sum(jnp.where(is2, e2, 0.0), axis=-1, keepdims=True) \
        / jnp.sum(e2, axis=-1, keepdims=True)

    oh1_ref[...] = is1.astype(jnp.float32)
    oh2_ref[...] = is2.astype(jnp.float32)
    m1_ref[...] = mult1
    m2_ref[...] = mult2


def _run_router(x, gate_w):
    return pl.pallas_call(
        _router_body,
        grid=(NT,),
        in_specs=[
            pl.BlockSpec((T, H), lambda t: (t, 0)),
            pl.BlockSpec((E, H), lambda t: (0, 0)),
        ],
        out_specs=[
            pl.BlockSpec((T, E), lambda t: (t, 0)),
            pl.BlockSpec((T, E), lambda t: (t, 0)),
            pl.BlockSpec((T, E), lambda t: (t, 0)),
            pl.BlockSpec((T, 1), lambda t: (t, 0)),
            pl.BlockSpec((T, 1), lambda t: (t, 0)),
        ],
        out_shape=[
            jax.ShapeDtypeStruct((N, E), jnp.float32),
            jax.ShapeDtypeStruct((N, E), jnp.float32),
            jax.ShapeDtypeStruct((N, E), jnp.float32),
            jax.ShapeDtypeStruct((N, 1), jnp.float32),
            jax.ShapeDtypeStruct((N, 1), jnp.float32),
        ],
    )(x, gate_w)


_CHUNK = 512
_NCH = N2 // _CHUNK


def _dispatch_body(oh1_ref, oh2_ref, pos_ref, te_ref, gmap_ref, slot_ref,
                   nde_ref):
    a = jnp.concatenate([oh1_ref[...], oh2_ref[...]], axis=0)

    r = lax.broadcasted_iota(jnp.int32, (_CHUNK, _CHUNK), 0)
    c = lax.broadcasted_iota(jnp.int32, (_CHUNK, _CHUNK), 1)
    ltri = (r >= c).astype(jnp.float32)
    carry = jnp.zeros((1, E), jnp.float32)
    parts = []
    for ci in range(_NCH):
        blk = a[ci * _CHUNK:(ci + 1) * _CHUNK]
        local = lax.dot_general(ltri, blk, (((1,), (0,)), ((), ())),
                                preferred_element_type=jnp.float32)
        parts.append(local + carry)
        carry = carry + jnp.sum(blk, axis=0, keepdims=True)
    c_incl = jnp.concatenate(parts, axis=0)

    counts = carry
    pc = jnp.floor((counts + (T - 1)) * (1.0 / T)) * T
    ntiles = jnp.sum(pc) * (1.0 / T)

    er = lax.broadcasted_iota(jnp.int32, (E, E), 0)
    ec = lax.broadcasted_iota(jnp.int32, (E, E), 1)
    ident = (er == ec).astype(jnp.float32)
    strict_u = (er < ec).astype(jnp.float32)
    base_row = lax.dot_general(pc, strict_u, (((1,), (0,)), ((), ())),
                               preferred_element_type=jnp.float32)
    base_col = lax.dot_general(ident, base_row, (((1,), (1,)), ((), ())),
                               preferred_element_type=jnp.float32)

    rank = jnp.sum(a * c_incl, axis=1, keepdims=True) - 1.0
    base_of_j = jnp.sum(a * base_row, axis=1, keepdims=True)
    pos_ref[...] = (rank + base_of_j).astype(jnp.int32)

    baseT_col = base_col * (1.0 / T)
    giota = lax.broadcasted_iota(jnp.int32, (1, G), 1).astype(jnp.float32)
    gvals = jnp.minimum(giota, ntiles - 1.0)
    cmp = (baseT_col <= gvals).astype(jnp.float32)
    te = jnp.sum(cmp, axis=0, keepdims=True) - 1.0
    te_ref[...] = te.astype(jnp.int32)
    gmap_ref[...] = gvals.astype(jnp.int32)

    ne_row = (counts >= 0.5).astype(jnp.float32)
    ne_col = lax.dot_general(ident, ne_row, (((1,), (1,)), ((), ())),
                             preferred_element_type=jnp.float32)
    changes = jnp.sum(ne_col * cmp, axis=0, keepdims=True)
    slot = changes - 1.0
    slot_ref[...] = (slot - 2.0 * jnp.floor(slot * 0.5)).astype(jnp.int32)

    pc_col = lax.dot_general(ident, pc, (((1,), (1,)), ((), ())),
                             preferred_element_type=jnp.float32)
    pcT_col = pc_col * (1.0 / T)
    endT_col = baseT_col + pcT_col
    oh = cmp * (gvals < endT_col).astype(jnp.float32)
    seg_end = jnp.sum(oh * endT_col, axis=0, keepdims=True)
    nxt_tile = jnp.minimum(seg_end, ntiles - 1.0)
    nde = jnp.sum((baseT_col <= nxt_tile).astype(jnp.float32),
                  axis=0, keepdims=True) - 1.0
    nde_ref[...] = nde.astype(jnp.int32)


def _run_dispatch(oh1, oh2):
    return pl.pallas_call(
        _dispatch_body,
        grid=(1,),
        in_specs=[
            pl.BlockSpec((N, E), lambda i: (0, 0)),
            pl.BlockSpec((N, E), lambda i: (0, 0)),
        ],
        out_specs=[
            pl.BlockSpec((N2, 1), lambda i: (0, 0)),
            pl.BlockSpec((1, G), lambda i: (0, 0)),
            pl.BlockSpec((1, G), lambda i: (0, 0)),
            pl.BlockSpec((1, G), lambda i: (0, 0)),
            pl.BlockSpec((1, G), lambda i: (0, 0)),
        ],
        out_shape=[
            jax.ShapeDtypeStruct((N2, 1), jnp.int32),
            jax.ShapeDtypeStruct((1, G), jnp.int32),
            jax.ShapeDtypeStruct((1, G), jnp.int32),
            jax.ShapeDtypeStruct((1, G), jnp.int32),
            jax.ShapeDtypeStruct((1, G), jnp.int32),
        ],
    )(oh1, oh2)


_ROWCHUNK = 32


def _make_sc_scatter():
    mesh = plsc.VectorSubcoreMesh(core_axis_name="c", subcore_axis_name="s")
    info = plsc.get_sparse_core_info()
    nc, ns = info.num_cores, info.num_subcores
    nw = nc * ns
    per_w = N // nw
    nch = per_w // _ROWCHUNK

    @functools.partial(
        pl.kernel,
        mesh=mesh,
        out_type=jax.ShapeDtypeStruct((SP, H), jnp.float32),
        scratch_types=[
            pltpu.VMEM((_ROWCHUNK,), jnp.int32),
            pltpu.VMEM((_ROWCHUNK,), jnp.int32),
            pltpu.VMEM((_ROWCHUNK, H), jnp.float32),
            pltpu.SemaphoreType.DMA,
        ],
    )
    def scatter_kernel(x_hbm, pos_hbm, xs_hbm, idx1_v, idx2_v, rows_v, sem):
        wid = lax.axis_index("s") * nc + lax.axis_index("c")
        base = wid * per_w
        for ch in range(nch):
            o = base + ch * _ROWCHUNK
            pltpu.sync_copy(x_hbm.at[pl.ds(o, _ROWCHUNK)], rows_v)
            pltpu.sync_copy(pos_hbm.at[pl.ds(o, _ROWCHUNK)], idx1_v)
            pltpu.sync_copy(pos_hbm.at[pl.ds(N + o, _ROWCHUNK)], idx2_v)
            pltpu.async_copy(rows_v, xs_hbm.at[idx1_v], sem).wait()
            pltpu.async_copy(rows_v, xs_hbm.at[idx2_v], sem).wait()

    return scatter_kernel


_WCH = 4

def _ffn_body(te_ref, gmap_ref, slot_ref, nde_ref, xs_ref,
              w1_hbm, w3_hbm, w2_hbm,
              ys_ref, w1s, w3s, w2s, w1b, w3b, w2b, s1, s3, s2):
    g = pl.program_id(0)
    cur = te_ref[g]
    slot = slot_ref[g]
    prv = jnp.where(g == 0, -1, te_ref[jnp.maximum(g - 1, 0)])
    nde = nde_ref[g]

    def _w13_copies(e):
        cps = []
        fc = F // _WCH
        for k in range(_WCH):
            cps.append(pltpu.make_async_copy(
                w1_hbm.at[e, pl.ds(k * fc, fc)], w1s.at[pl.ds(k * fc, fc)],
                s1.at[k]))
            cps.append(pltpu.make_async_copy(
                w3_hbm.at[e, pl.ds(k * fc, fc)], w3s.at[pl.ds(k * fc, fc)],
                s3.at[k]))
        return cps

    def _w2_copy(e, k):
        fc = F // _WCH
        return pltpu.make_async_copy(
            w2_hbm.at[e, :, pl.ds(k * fc, fc)], w2s.at[:, pl.ds(k * fc, fc)],
            s2.at[k])

    @pl.when(g == 0)
    def _():
        for cp in _w13_copies(cur):
            cp.start()
        for k in range(_WCH):
            _w2_copy(cur, k).start()

    @pl.when(cur != prv)
    def _():
        for cp in _w13_copies(cur):
            cp.wait()
        fc = F // _WCH
        for k in range(_WCH):
            w1b[pl.ds(k * fc, fc), :] = \
                w1s[pl.ds(k * fc, fc), :].astype(jnp.bfloat16)
            w3b[pl.ds(k * fc, fc), :] = \
                w3s[pl.ds(k * fc, fc), :].astype(jnp.bfloat16)

        @pl.when(nde != cur)
        def _():
            for cp in _w13_copies(nde):
                cp.start()

    @pl.when(gmap_ref[g] == g)
    def _():
        xs = xs_ref[...].astype(jnp.bfloat16)
        acc = None
        kw = _WCH // FK
        for fk in range(FK):
            @pl.when(cur != prv)
            def _(fk=fk):
                for k in range(kw * fk, kw * (fk + 1)):
                    _w2_copy(cur, k).wait()
            w1c = w1b[pl.ds(fk * FC, FC), :]
            w3c = w3b[pl.ds(fk * FC, FC), :]
            w2c = w2s[:, pl.ds(fk * FC, FC)]
            h1 = lax.dot_general(xs, w1c, (((1,), (1,)), ((), ())),
                                 preferred_element_type=jnp.float32)
            h3 = lax.dot_general(xs, w3c, (((1,), (1,)), ((), ())),
                                 preferred_element_type=jnp.float32)
            act = (h1 / (1.0 + jnp.exp(-h1))) * h3
            part = lax.dot_general(act, w2c, (((1,), (1,)), ((), ())),
                                   preferred_element_type=jnp.float32)
            acc = part if acc is None else acc + part
        ys_ref[...] = acc

    @pl.when(jnp.logical_and(g + 1 < G,
                             te_ref[jnp.minimum(g + 1, G - 1)] != cur))
    def _():
        for k in range(_WCH):
            _w2_copy(nde, k).start()


def _run_ffn(xs, te, gmap, slot, nde, w1, w3, w2):
    grid_spec = pltpu.PrefetchScalarGridSpec(
        num_scalar_prefetch=4,
        grid=(G,),
        in_specs=[
            pl.BlockSpec((T, H), lambda g, te, gm, sl, nd: (gm[g], 0)),
            pl.BlockSpec(memory_space=pltpu.MemorySpace.HBM),
            pl.BlockSpec(memory_space=pltpu.MemorySpace.HBM),
            pl.BlockSpec(memory_space=pltpu.MemorySpace.HBM),
        ],
        out_specs=pl.BlockSpec((T, H), lambda g, te, gm, sl, nd: (gm[g], 0)),
        scratch_shapes=[
            pltpu.VMEM((F, H), jnp.float32),
            pltpu.VMEM((F, H), jnp.float32),
            pltpu.VMEM((H, F), jnp.float32),
            pltpu.VMEM((F, H), jnp.bfloat16),
            pltpu.VMEM((F, H), jnp.bfloat16),
            pltpu.VMEM((H, F), jnp.bfloat16),
            pltpu.SemaphoreType.DMA((_WCH,)),
            pltpu.SemaphoreType.DMA((_WCH,)),
            pltpu.SemaphoreType.DMA((_WCH,)),
        ],
    )
    return pl.pallas_call(
        _ffn_body,
        grid_spec=grid_spec,
        out_shape=jax.ShapeDtypeStruct((SP, H), jnp.float32),
    )(te, gmap, slot, nde, xs, w1, w3, w2)


def _make_sc_gather():
    mesh = plsc.VectorSubcoreMesh(core_axis_name="c", subcore_axis_name="s")
    info = plsc.get_sparse_core_info()
    nc, ns = info.num_cores, info.num_subcores
    nw = nc * ns
    per_w = N // nw
    nch = per_w // _ROWCHUNK

    @functools.partial(
        pl.kernel,
        mesh=mesh,
        out_type=[
            jax.ShapeDtypeStruct((N, H), jnp.float32),
            jax.ShapeDtypeStruct((N, H), jnp.float32),
        ],
        scratch_types=[
            pltpu.VMEM((_ROWCHUNK,), jnp.int32),
            pltpu.VMEM((_ROWCHUNK,), jnp.int32),
            pltpu.VMEM((_ROWCHUNK, H), jnp.float32),
            pltpu.VMEM((_ROWCHUNK, H), jnp.float32),
            pltpu.SemaphoreType.DMA,
        ],
    )
    def gather_kernel(ys_hbm, pos_hbm, y1_hbm, y2_hbm,
                      idx1_v, idx2_v, rows1_v, rows2_v, sem):
        wid = lax.axis_index("s") * nc + lax.axis_index("c")
        base = wid * per_w
        for ch in range(nch):
            o = base + ch * _ROWCHUNK
            pltpu.sync_copy(pos_hbm.at[pl.ds(o, _ROWCHUNK)], idx1_v)
            pltpu.sync_copy(pos_hbm.at[pl.ds(N + o, _ROWCHUNK)], idx2_v)
            pltpu.async_copy(ys_hbm.at[idx1_v], rows1_v, sem).wait()
            pltpu.async_copy(ys_hbm.at[idx2_v], rows2_v, sem).wait()
            pltpu.sync_copy(rows1_v, y1_hbm.at[pl.ds(o, _ROWCHUNK)])
            pltpu.sync_copy(rows2_v, y2_hbm.at[pl.ds(o, _ROWCHUNK)])

    return gather_kernel


def _combine_body(y1_ref, y2_ref, m1_ref, m2_ref, out_ref):
    y1 = y1_ref[...].astype(jnp.float32)
    y2 = y2_ref[...].astype(jnp.float32)
    out_ref[...] = y1 * m1_ref[...] + y2 * m2_ref[...]


def _run_combine(y1, y2, m1, m2):
    return pl.pallas_call(
        _combine_body,
        grid=(NT,),
        in_specs=[
            pl.BlockSpec((T, H), lambda t: (t, 0)),
            pl.BlockSpec((T, H), lambda t: (t, 0)),
            pl.BlockSpec((T, 1), lambda t: (t, 0)),
            pl.BlockSpec((T, 1), lambda t: (t, 0)),
        ],
        out_specs=pl.BlockSpec((T, H), lambda t: (t, 0)),
        out_shape=jax.ShapeDtypeStruct((N, H), jnp.float32),
    )(y1, y2, m1, m2)


def kernel(hidden_states, gate_w, w1, w2, w3):
    B, S, Hd = hidden_states.shape
    x = hidden_states.reshape(-1, Hd)
    logits, oh1, oh2, m1, m2 = _run_router(x, gate_w)
    pos2, te2, gmap2, slot2, nde2 = _run_dispatch(oh1, oh2)
    pos = pos2.reshape(N2)
    te = te2.reshape(G)
    gmap = gmap2.reshape(G)
    slot = slot2.reshape(G)
    nde = nde2.reshape(G)
    xs = _make_sc_scatter()(x, pos)
    ys = _run_ffn(xs, te, gmap, slot, nde, w1, w3, w2)
    y1, y2 = _make_sc_gather()(ys, pos)
    out = _run_combine(y1, y2, m1, m2)
    return out.reshape(B, S, Hd), logits
